# baseline (speedup 1.0000x reference)
import functools

import jax
import jax.numpy as jnp
from jax import lax
from jax.experimental import pallas as pl
from jax.experimental.pallas import tpu as pltpu
from jax.experimental.pallas import tpu_sc as plsc

N = 10000
NP = 10240
E = 320000
D = 128
DH = D // 2
NC = 2
NS = 16
WIN = 80
EPS = E // NS
NWIN = EPS // WIN
ROWS_PT = NP // NS

_MESH = plsc.VectorSubcoreMesh(core_axis_name="c", subcore_axis_name="s")

_MAGIC = 0x5F3759DF


@functools.partial(
    pl.kernel,
    out_type=(
        jax.ShapeDtypeStruct((NC, NP, DH), jnp.float32),
        jax.ShapeDtypeStruct((2 * NP,), jnp.float32),
        jax.ShapeDtypeStruct((NP,), jnp.float32),
    ),
    mesh=_MESH,
    scratch_types=[
        pltpu.VMEM((NWIN, WIN), jnp.int32),
        pltpu.VMEM((NWIN, WIN), jnp.int32),
        pltpu.VMEM((NWIN, WIN), jnp.float32),
        pltpu.VMEM((WIN,), jnp.float32),
        pltpu.VMEM((ROWS_PT,), jnp.float32),
        [pltpu.VMEM((WIN, DH), jnp.float32) for _ in range(5)],
        [pltpu.VMEM((WIN,), jnp.float32) for _ in range(5)],
        pltpu.VMEM_SHARED((NP, DH), jnp.float32),
        pltpu.VMEM_SHARED((NP,), jnp.float32),
        pltpu.VMEM_SHARED((NP,), jnp.float32),
        [pltpu.SemaphoreType.DMA for _ in range(5)],
        [pltpu.SemaphoreType.DMA for _ in range(5)],
        pltpu.SemaphoreType.DMA,
    ],
    compiler_params=pltpu.CompilerParams(use_tc_tiling_on_sc=False),
)
def _sc_kernel(xs_hbm, src_hbm, dst_hbm, w_hbm, zrows_hbm, zdeg_hbm, ones_hbm,
               parts_hbm, norm_hbm, degi_hbm,
               src_v, dst_v, w_v, ones_v, nrm_v, bufs, nbufs,
               agg_sh, dego_sh, degi_sh, gsems, ssems, dsem):
    cid = lax.axis_index("c")
    sid = lax.axis_index("s")

    pltpu.sync_copy(src_hbm.at[sid], src_v)
    pltpu.sync_copy(dst_hbm.at[sid], dst_v)
    pltpu.sync_copy(w_hbm.at[sid], w_v)
    pltpu.sync_copy(ones_hbm, ones_v)
    pltpu.sync_copy(zrows_hbm, agg_sh.at[pl.ds(sid * ROWS_PT, ROWS_PT)])
    pltpu.sync_copy(zdeg_hbm, dego_sh.at[pl.ds(sid * ROWS_PT, ROWS_PT)])
    pltpu.sync_copy(zdeg_hbm, degi_sh.at[pl.ds(sid * ROWS_PT, ROWS_PT)])
    plsc.subcore_barrier()

    def dbatch(bi, carry):
        for q in range(5):
            j = bi * 5 + q
            pltpu.async_copy(ones_v, dego_sh.at[src_v.at[j]], dsem, add=True)
            pltpu.async_copy(ones_v, degi_sh.at[dst_v.at[j]], dsem, add=True)
        for q in range(5):
            j = bi * 5 + q
            pltpu.make_async_copy(ones_v, dego_sh.at[src_v.at[j]],
                                  dsem).wait()
            pltpu.make_async_copy(ones_v, degi_sh.at[dst_v.at[j]],
                                  dsem).wait()
        return carry

    lax.fori_loop(0, NWIN // 5, dbatch, 0)
    plsc.subcore_barrier()

    pltpu.sync_copy(dego_sh.at[pl.ds(sid * ROWS_PT, ROWS_PT)], nrm_v)
    for k in range(ROWS_PT // 16):
        sl = pl.ds(k * 16, 16)
        d = jnp.maximum(nrm_v[sl], 1.0)
        i = lax.bitcast_convert_type(d, jnp.int32)
        y = lax.bitcast_convert_type(_MAGIC - (i >> 1), jnp.float32)
        y = y * (1.5 - 0.5 * d * y * y)
        y = y * (1.5 - 0.5 * d * y * y)
        y = y * (1.5 - 0.5 * d * y * y)
        nrm_v[sl] = y
    pltpu.sync_copy(nrm_v, norm_hbm.at[pl.ds(cid * NP + sid * ROWS_PT,
                                             ROWS_PT)])

    @pl.when(cid == 0)
    def _():
        pltpu.sync_copy(degi_sh.at[pl.ds(sid * ROWS_PT, ROWS_PT)],
                        degi_hbm.at[pl.ds(sid * ROWS_PT, ROWS_PT)])

    coff = jnp.full((16,), cid * NP, dtype=jnp.int32)

    def fix(i, carry):
        j = i // (WIN // 16)
        g = i - j * (WIN // 16)
        sl = pl.ds(g * 16, 16)
        src_v[j, sl] = src_v[j, sl] + coff
        return carry

    lax.fori_loop(0, NWIN * (WIN // 16), fix, 0)
    plsc.subcore_barrier()

    def cbatch(bi, carry):
        for q in range(5):
            pltpu.async_copy(norm_hbm.at[src_v.at[bi * 5 + q]], nbufs[q],
                             dsem)
        for q in range(5):
            pltpu.make_async_copy(norm_hbm.at[src_v.at[bi * 5 + q]],
                                  nbufs[q], dsem).wait()
        for q in range(5):
            j = bi * 5 + q
            for g in range(WIN // 16):
                sl = pl.ds(g * 16, 16)
                w_v[j, sl] = w_v[j, sl] * nbufs[q][sl]
        return carry

    lax.fori_loop(0, NWIN // 5, cbatch, 0)

    def scale(buf, j):
        for g in range(WIN // 16):
            w16 = w_v[j, pl.ds(g * 16, 16)]
            for l in range(16):
                w = w16[l]
                for k in range(DH // 16):
                    sl = pl.ds(k * 16, 16)
                    buf[g * 16 + l, sl] = buf[g * 16 + l, sl] * w

    for q in range(3):
        pltpu.async_copy(xs_hbm.at[src_v.at[q]], bufs[q], gsems[q])

    def slot(q, base):
        j = base + q
        pltpu.make_async_copy(xs_hbm.at[src_v.at[j]], bufs[q],
                              gsems[q]).wait()
        scale(bufs[q], j)
        pltpu.async_copy(bufs[q], agg_sh.at[dst_v.at[j]], ssems[q], add=True)
        qn = (q + 3) % 5

        @pl.when(j >= 2)
        def _():
            pltpu.make_async_copy(bufs[qn], agg_sh.at[dst_v.at[j - 2]],
                                  ssems[qn]).wait()

        @pl.when(j + 3 < NWIN)
        def _():
            pltpu.async_copy(xs_hbm.at[src_v.at[j + 3]], bufs[qn], gsems[qn])

    def block(i, carry):
        base = i * 5
        for q in range(5):
            slot(q, base)
        return carry

    lax.fori_loop(0, NWIN // 5, block, 0)
    pltpu.make_async_copy(bufs[3], agg_sh.at[dst_v.at[NWIN - 2]],
                          ssems[3]).wait()
    pltpu.make_async_copy(bufs[4], agg_sh.at[dst_v.at[NWIN - 1]],
                          ssems[4]).wait()
    plsc.subcore_barrier()
    pltpu.sync_copy(agg_sh.at[pl.ds(sid * ROWS_PT, ROWS_PT)],
                    parts_hbm.at[cid, pl.ds(sid * ROWS_PT, ROWS_PT)])


BLK = 1280


def _out_body(p_ref, w_ref, degi_ref, b_ref, o_ref):
    p = p_ref[...]
    a = jnp.concatenate([p[0], p[1]], axis=1)
    r = jnp.dot(a, w_ref[...], preferred_element_type=jnp.float32)
    d = degi_ref[...]
    norm = lax.rsqrt(jnp.maximum(d[0], 1.0))
    o_ref[...] = r * norm[:, None] + b_ref[...]


_out_call = pl.pallas_call(
    _out_body,
    grid=(NP // BLK,),
    in_specs=[
        pl.BlockSpec((NC, BLK, DH), lambda i: (0, i, 0)),
        pl.BlockSpec((D, D), lambda i: (0, 0)),
        pl.BlockSpec((1, BLK), lambda i: (0, i)),
        pl.BlockSpec((1, D), lambda i: (0, 0)),
    ],
    out_specs=pl.BlockSpec((BLK, D), lambda i: (i, 0)),
    out_shape=jax.ShapeDtypeStruct((NP, D), jnp.float32),
)


def kernel(node_embedding, edge_embedding, edge_index, W, b):
    ei = edge_index.astype(jnp.int32)
    src_a = ei[0].reshape(NS, NWIN, WIN)
    dst_a = ei[1].reshape(NS, NWIN, WIN)
    wts = edge_embedding.astype(jnp.float32).reshape(NS, NWIN, WIN)
    ones = jnp.ones((WIN,), jnp.float32)
    zdeg = jnp.zeros((ROWS_PT,), jnp.float32)
    zrows = jnp.zeros((ROWS_PT, DH), jnp.float32)
    xs = (jnp.zeros((NC * NP, DH), jnp.float32)
          .at[:N].set(node_embedding[:, :DH])
          .at[NP:NP + N].set(node_embedding[:, DH:]))

    parts, _, degi = _sc_kernel(xs, src_a, dst_a, wts, zrows, zdeg, ones)
    return _out_call(parts, W, degi.reshape(1, NP), b.reshape(1, D))[:N]


# speedup vs baseline: 1.2486x; 1.2486x over previous
import functools

import jax
import jax.numpy as jnp
from jax import lax
from jax.experimental import pallas as pl
from jax.experimental.pallas import tpu as pltpu
from jax.experimental.pallas import tpu_sc as plsc

N = 10000
NP = 10240
E = 320000
D = 128
DH = D // 2
NC = 2
NS = 16
NW = NC * NS
WIN = 80
EPW = E // NW
NWIN_D = EPW // WIN
EPS = E // NS
NWIN_A = EPS // WIN
ROWS_PT = NP // NS
DEGW = (2 * NP) // NS

_MESH = plsc.VectorSubcoreMesh(core_axis_name="c", subcore_axis_name="s")


@functools.partial(
    pl.kernel,
    out_type=jax.ShapeDtypeStruct((NC, 2 * NP), jnp.float32),
    mesh=_MESH,
    scratch_types=[
        pltpu.VMEM((2 * NWIN_D, WIN), jnp.int32),
        pltpu.VMEM((WIN,), jnp.float32),
        pltpu.VMEM_SHARED((2 * NP,), jnp.float32),
        pltpu.SemaphoreType.DMA,
    ],
)
def _deg_kernel(idx_hbm, ones_hbm, zdeg_hbm, degp_hbm, idx_v, ones_v, deg_sh,
                dsem):
    cid = lax.axis_index("c")
    sid = lax.axis_index("s")
    tid = cid * NS + sid
    pltpu.sync_copy(zdeg_hbm, deg_sh.at[pl.ds(sid * DEGW, DEGW)])
    pltpu.sync_copy(idx_hbm.at[tid], idx_v)
    pltpu.sync_copy(ones_hbm, ones_v)
    plsc.subcore_barrier()

    def batch(bi, carry):
        for q in range(10):
            pltpu.async_copy(ones_v, deg_sh.at[idx_v.at[bi * 10 + q]], dsem,
                             add=True)
        for q in range(10):
            pltpu.make_async_copy(ones_v, deg_sh.at[idx_v.at[bi * 10 + q]],
                                  dsem).wait()
        return carry

    lax.fori_loop(0, (2 * NWIN_D) // 10, batch, 0)
    plsc.subcore_barrier()
    pltpu.sync_copy(deg_sh.at[pl.ds(sid * DEGW, DEGW)],
                    degp_hbm.at[cid, pl.ds(sid * DEGW, DEGW)])


@functools.partial(
    pl.kernel,
    out_type=jax.ShapeDtypeStruct((NC, NP, DH), jnp.float32),
    mesh=_MESH,
    scratch_types=[
        pltpu.VMEM((NWIN_A, WIN), jnp.int32),
        pltpu.VMEM((NWIN_A, WIN), jnp.int32),
        pltpu.VMEM((NWIN_A, WIN), jnp.float32),
        [pltpu.VMEM((WIN, DH), jnp.float32) for _ in range(5)],
        pltpu.VMEM_SHARED((NP, DH), jnp.float32),
        [pltpu.SemaphoreType.DMA for _ in range(5)],
        [pltpu.SemaphoreType.DMA for _ in range(5)],
    ],
    compiler_params=pltpu.CompilerParams(use_tc_tiling_on_sc=False),
)
def _agg_kernel(feat_hbm, src_hbm, dst_hbm, w_hbm, zrows_hbm, parts_hbm,
                src_v, dst_v, w_v, bufs, agg_sh, gsems, ssems):
    cid = lax.axis_index("c")
    sid = lax.axis_index("s")
    pltpu.sync_copy(zrows_hbm, agg_sh.at[pl.ds(sid * ROWS_PT, ROWS_PT)])
    pltpu.sync_copy(src_hbm.at[sid], src_v)
    pltpu.sync_copy(dst_hbm.at[sid], dst_v)
    pltpu.sync_copy(w_hbm.at[sid], w_v)

    coff = jnp.full((16,), cid * NP, dtype=jnp.int32)

    def fix(i, carry):
        j = i // (WIN // 16)
        g = i - j * (WIN // 16)
        sl = pl.ds(g * 16, 16)
        src_v[j, sl] = src_v[j, sl] + coff
        return carry

    lax.fori_loop(0, NWIN_A * (WIN // 16), fix, 0)
    plsc.subcore_barrier()

    def scale(buf, j):
        for g in range(WIN // 16):
            w16 = w_v[j, pl.ds(g * 16, 16)]
            for l in range(16):
                w = w16[l]
                for k in range(DH // 16):
                    sl = pl.ds(k * 16, 16)
                    buf[g * 16 + l, sl] = buf[g * 16 + l, sl] * w

    H = WIN // 2

    def gstart(j, q):
        pltpu.async_copy(feat_hbm.at[src_v.at[j, pl.ds(0, H)]],
                         bufs[q].at[pl.ds(0, H)], gsems[q])
        pltpu.async_copy(feat_hbm.at[src_v.at[j, pl.ds(H, H)]],
                         bufs[q].at[pl.ds(H, H)], gsems[q])

    def gwait(j, q):
        pltpu.make_async_copy(feat_hbm.at[src_v.at[j, pl.ds(0, H)]],
                              bufs[q].at[pl.ds(0, H)], gsems[q]).wait()
        pltpu.make_async_copy(feat_hbm.at[src_v.at[j, pl.ds(H, H)]],
                              bufs[q].at[pl.ds(H, H)], gsems[q]).wait()

    for q in range(3):
        gstart(q, q)

    def slot(q, base):
        j = base + q
        gwait(j, q)
        scale(bufs[q], j)
        pltpu.async_copy(bufs[q], agg_sh.at[dst_v.at[j]], ssems[q], add=True)
        qn = (q + 3) % 5

        @pl.when(j >= 2)
        def _():
            pltpu.make_async_copy(bufs[qn], agg_sh.at[dst_v.at[j - 2]],
                                  ssems[qn]).wait()

        @pl.when(j + 3 < NWIN_A)
        def _():
            gstart(j + 3, qn)

    def block(i, carry):
        base = i * 5
        for q in range(5):
            slot(q, base)
        return carry

    lax.fori_loop(0, NWIN_A // 5, block, 0)
    pltpu.make_async_copy(bufs[3], agg_sh.at[dst_v.at[NWIN_A - 2]],
                          ssems[3]).wait()
    pltpu.make_async_copy(bufs[4], agg_sh.at[dst_v.at[NWIN_A - 1]],
                          ssems[4]).wait()
    plsc.subcore_barrier()
    pltpu.sync_copy(agg_sh.at[pl.ds(sid * ROWS_PT, ROWS_PT)],
                    parts_hbm.at[cid, pl.ds(sid * ROWS_PT, ROWS_PT)])


BLK = 1280


def _feat_body(x_ref, degp_ref, feat_ref):
    d = degp_ref[...]
    norm = lax.rsqrt(jnp.maximum(d[0] + d[1], 1.0))
    xb = x_ref[...] * norm[:, None]
    feat_ref[0] = xb[:, :DH]
    feat_ref[1] = xb[:, DH:]


_feat_call = pl.pallas_call(
    _feat_body,
    grid=(NP // BLK,),
    in_specs=[
        pl.BlockSpec((BLK, D), lambda i: (i, 0)),
        pl.BlockSpec((2, BLK), lambda i: (0, i)),
    ],
    out_specs=pl.BlockSpec((NC, BLK, DH), lambda i: (0, i, 0)),
    out_shape=jax.ShapeDtypeStruct((NC, NP, DH), jnp.float32),
)


def _out_body(p_ref, w_ref, degp_ref, b_ref, o_ref):
    p = p_ref[...]
    a = jnp.concatenate([p[0], p[1]], axis=1)
    r = jnp.dot(a, w_ref[...], preferred_element_type=jnp.float32)
    d = degp_ref[...]
    norm = lax.rsqrt(jnp.maximum(d[0] + d[1], 1.0))
    o_ref[...] = r * norm[:, None] + b_ref[...]


_out_call = pl.pallas_call(
    _out_body,
    grid=(NP // BLK,),
    in_specs=[
        pl.BlockSpec((NC, BLK, DH), lambda i: (0, i, 0)),
        pl.BlockSpec((D, D), lambda i: (0, 0)),
        pl.BlockSpec((2, BLK), lambda i: (0, NP // BLK + i)),
        pl.BlockSpec((1, D), lambda i: (0, 0)),
    ],
    out_specs=pl.BlockSpec((BLK, D), lambda i: (i, 0)),
    out_shape=jax.ShapeDtypeStruct((NP, D), jnp.float32),
)


def kernel(node_embedding, edge_embedding, edge_index, W, b):
    ei = edge_index.astype(jnp.int32)
    src_d = ei[0].reshape(NW, NWIN_D, WIN)
    dst_d = ei[1].reshape(NW, NWIN_D, WIN)
    deg_idx = jnp.concatenate([src_d, dst_d + NP], axis=1)
    src_a = ei[0].reshape(NS, NWIN_A, WIN)
    dst_a = ei[1].reshape(NS, NWIN_A, WIN)
    wts = edge_embedding.astype(jnp.float32).reshape(NS, NWIN_A, WIN)
    ones = jnp.ones((WIN,), jnp.float32)
    zdeg = jnp.zeros((DEGW,), jnp.float32)
    zrows = jnp.zeros((ROWS_PT, DH), jnp.float32)
    x_pad = jnp.zeros((NP, D), jnp.float32).at[:N].set(node_embedding)

    degp = _deg_kernel(deg_idx, ones, zdeg)
    feat = _feat_call(x_pad, degp).reshape(2 * NP, DH)
    parts = _agg_kernel(feat, src_a, dst_a, wts, zrows)
    return _out_call(parts, W, degp, b.reshape(1, D))[:N]


# speedup vs baseline: 1.2698x; 1.0169x over previous
import functools

import jax
import jax.numpy as jnp
from jax import lax
from jax.experimental import pallas as pl
from jax.experimental.pallas import tpu as pltpu
from jax.experimental.pallas import tpu_sc as plsc

N = 10000
NP = 10240
E = 320000
D = 128
DH = D // 2
NC = 2
NS = 16
NW = NC * NS
WIN = 80
EPW = E // NW
NWIN_D = EPW // WIN
EPS = E // NS
NWIN_A = EPS // WIN
ROWS_PT = NP // NS
DEGW = (2 * NP) // NS

_MESH = plsc.VectorSubcoreMesh(core_axis_name="c", subcore_axis_name="s")


@functools.partial(
    pl.kernel,
    out_type=jax.ShapeDtypeStruct((NC, 2 * NP), jnp.float32),
    mesh=_MESH,
    scratch_types=[
        pltpu.VMEM((NWIN_D, WIN), jnp.int32),
        pltpu.VMEM((NWIN_D, WIN), jnp.int32),
        pltpu.VMEM((WIN,), jnp.float32),
        pltpu.VMEM_SHARED((NP,), jnp.float32),
        pltpu.VMEM_SHARED((NP,), jnp.float32),
        pltpu.SemaphoreType.DMA,
    ],
)
def _deg_kernel(src_hbm, dst_hbm, ones_hbm, zdeg_hbm, degp_hbm,
                srci_v, dsti_v, ones_v, dego_sh, degi_sh, dsem):
    cid = lax.axis_index("c")
    sid = lax.axis_index("s")
    tid = cid * NS + sid
    pltpu.sync_copy(zdeg_hbm, dego_sh.at[pl.ds(sid * (NP // NS), NP // NS)])
    pltpu.sync_copy(zdeg_hbm, degi_sh.at[pl.ds(sid * (NP // NS), NP // NS)])
    pltpu.sync_copy(src_hbm.at[tid], srci_v)
    pltpu.sync_copy(dst_hbm.at[tid], dsti_v)
    pltpu.sync_copy(ones_hbm, ones_v)
    plsc.subcore_barrier()

    def batch(bi, carry):
        for q in range(5):
            j = bi * 5 + q
            pltpu.async_copy(ones_v, dego_sh.at[srci_v.at[j]], dsem, add=True)
            pltpu.async_copy(ones_v, degi_sh.at[dsti_v.at[j]], dsem, add=True)
        for q in range(5):
            j = bi * 5 + q
            pltpu.make_async_copy(ones_v, dego_sh.at[srci_v.at[j]],
                                  dsem).wait()
            pltpu.make_async_copy(ones_v, degi_sh.at[dsti_v.at[j]],
                                  dsem).wait()
        return carry

    lax.fori_loop(0, NWIN_D // 5, batch, 0)
    plsc.subcore_barrier()
    pltpu.sync_copy(dego_sh.at[pl.ds(sid * (NP // NS), NP // NS)],
                    degp_hbm.at[cid, pl.ds(sid * (NP // NS), NP // NS)])
    pltpu.sync_copy(degi_sh.at[pl.ds(sid * (NP // NS), NP // NS)],
                    degp_hbm.at[cid, pl.ds(NP + sid * (NP // NS), NP // NS)])


@functools.partial(
    pl.kernel,
    out_type=jax.ShapeDtypeStruct((NC, NP, DH), jnp.float32),
    mesh=_MESH,
    scratch_types=[
        pltpu.VMEM((NWIN_A, WIN), jnp.int32),
        pltpu.VMEM((NWIN_A, WIN), jnp.int32),
        pltpu.VMEM((NWIN_A, WIN), jnp.float32),
        [pltpu.VMEM((WIN, DH), jnp.float32) for _ in range(5)],
        pltpu.VMEM_SHARED((NP, DH), jnp.float32),
        [pltpu.SemaphoreType.DMA for _ in range(5)],
        [pltpu.SemaphoreType.DMA for _ in range(5)],
    ],
    compiler_params=pltpu.CompilerParams(use_tc_tiling_on_sc=False),
)
def _agg_kernel(feat_hbm, src_hbm, dst_hbm, w_hbm, zrows_hbm, parts_hbm,
                src_v, dst_v, w_v, bufs, agg_sh, gsems, ssems):
    cid = lax.axis_index("c")
    sid = lax.axis_index("s")
    pltpu.sync_copy(zrows_hbm, agg_sh.at[pl.ds(sid * ROWS_PT, ROWS_PT)])
    pltpu.sync_copy(src_hbm.at[sid], src_v)
    pltpu.sync_copy(dst_hbm.at[sid], dst_v)
    pltpu.sync_copy(w_hbm.at[sid], w_v)

    coff = jnp.full((16,), cid * NP, dtype=jnp.int32)

    def fix(i, carry):
        j = i // (WIN // 16)
        g = i - j * (WIN // 16)
        sl = pl.ds(g * 16, 16)
        src_v[j, sl] = src_v[j, sl] + coff
        return carry

    def fix3(i, carry):
        return fix(i, carry)

    lax.fori_loop(0, 3 * (WIN // 16), fix3, 0)
    for q in range(3):
        pltpu.async_copy(feat_hbm.at[src_v.at[q]], bufs[q], gsems[q])
    lax.fori_loop(3 * (WIN // 16), NWIN_A * (WIN // 16), fix, 0)
    plsc.subcore_barrier()

    def scale(buf, j):
        for g in range(WIN // 16):
            w16 = w_v[j, pl.ds(g * 16, 16)]
            for l in range(16):
                w = w16[l]
                for k in range(DH // 16):
                    sl = pl.ds(k * 16, 16)
                    buf[g * 16 + l, sl] = buf[g * 16 + l, sl] * w

    def slot(q, base):
        j = base + q
        pltpu.make_async_copy(feat_hbm.at[src_v.at[j]], bufs[q],
                              gsems[q]).wait()
        scale(bufs[q], j)
        pltpu.async_copy(bufs[q], agg_sh.at[dst_v.at[j]], ssems[q], add=True)
        qn = (q + 3) % 5

        @pl.when(j >= 2)
        def _():
            pltpu.make_async_copy(bufs[qn], agg_sh.at[dst_v.at[j - 2]],
                                  ssems[qn]).wait()

        @pl.when(j + 3 < NWIN_A)
        def _():
            pltpu.async_copy(feat_hbm.at[src_v.at[j + 3]], bufs[qn],
                             gsems[qn])

    def block(i, carry):
        base = i * 5
        for q in range(5):
            slot(q, base)
        return carry

    lax.fori_loop(0, NWIN_A // 5, block, 0)
    pltpu.make_async_copy(bufs[3], agg_sh.at[dst_v.at[NWIN_A - 2]],
                          ssems[3]).wait()
    pltpu.make_async_copy(bufs[4], agg_sh.at[dst_v.at[NWIN_A - 1]],
                          ssems[4]).wait()
    plsc.subcore_barrier()
    pltpu.sync_copy(agg_sh.at[pl.ds(sid * ROWS_PT, ROWS_PT)],
                    parts_hbm.at[cid, pl.ds(sid * ROWS_PT, ROWS_PT)])


BLK = 1280


def _feat_body(x_ref, degp_ref, feat_ref):
    d = degp_ref[...]
    norm = lax.rsqrt(jnp.maximum(d[0] + d[1], 1.0))
    xb = x_ref[...] * norm[:, None]
    feat_ref[0] = xb[:, :DH]
    feat_ref[1] = xb[:, DH:]


_feat_call = pl.pallas_call(
    _feat_body,
    grid=(NP // BLK,),
    in_specs=[
        pl.BlockSpec((BLK, D), lambda i: (i, 0)),
        pl.BlockSpec((2, BLK), lambda i: (0, i)),
    ],
    out_specs=pl.BlockSpec((NC, BLK, DH), lambda i: (0, i, 0)),
    out_shape=jax.ShapeDtypeStruct((NC, NP, DH), jnp.float32),
)


def _out_body(p_ref, w_ref, degp_ref, b_ref, o_ref):
    p = p_ref[...]
    a = jnp.concatenate([p[0], p[1]], axis=1)
    r = jnp.dot(a, w_ref[...], preferred_element_type=jnp.float32)
    d = degp_ref[...]
    norm = lax.rsqrt(jnp.maximum(d[0] + d[1], 1.0))
    o_ref[...] = r * norm[:, None] + b_ref[...]


_out_call = pl.pallas_call(
    _out_body,
    grid=(NP // BLK,),
    in_specs=[
        pl.BlockSpec((NC, BLK, DH), lambda i: (0, i, 0)),
        pl.BlockSpec((D, D), lambda i: (0, 0)),
        pl.BlockSpec((2, BLK), lambda i: (0, NP // BLK + i)),
        pl.BlockSpec((1, D), lambda i: (0, 0)),
    ],
    out_specs=pl.BlockSpec((BLK, D), lambda i: (i, 0)),
    out_shape=jax.ShapeDtypeStruct((NP, D), jnp.float32),
)


def kernel(node_embedding, edge_embedding, edge_index, W, b):
    ei = edge_index.astype(jnp.int32)
    src_d = ei[0].reshape(NW, NWIN_D, WIN)
    dst_d = ei[1].reshape(NW, NWIN_D, WIN)
    src_a = ei[0].reshape(NS, NWIN_A, WIN)
    dst_a = ei[1].reshape(NS, NWIN_A, WIN)
    wts = edge_embedding.astype(jnp.float32).reshape(NS, NWIN_A, WIN)
    ones = jnp.ones((WIN,), jnp.float32)
    zdeg = jnp.zeros((NP // NS,), jnp.float32)
    zrows = jnp.zeros((ROWS_PT, DH), jnp.float32)
    x_pad = jnp.zeros((NP, D), jnp.float32).at[:N].set(node_embedding)

    degp = _deg_kernel(src_d, dst_d, ones, zdeg)
    feat = _feat_call(x_pad, degp).reshape(2 * NP, DH)
    parts = _agg_kernel(feat, src_a, dst_a, wts, zrows)
    return _out_call(parts, W, degp, b.reshape(1, D))[:N]


# speedup vs baseline: 1.3107x; 1.0323x over previous
import functools

import jax
import jax.numpy as jnp
from jax import lax
from jax.experimental import pallas as pl
from jax.experimental.pallas import tpu as pltpu
from jax.experimental.pallas import tpu_sc as plsc

N = 10000
NP = 10240
E = 320000
D = 128
DH = D // 2
NC = 2
NS = 16
NW = NC * NS
WIN = 80
EPW = E // NW
NWIN_D = EPW // WIN
EPS = E // NS
NWIN_A = EPS // WIN
ROWS_PT = NP // NS
DEGW = (2 * NP) // NS

_MESH = plsc.VectorSubcoreMesh(core_axis_name="c", subcore_axis_name="s")


@functools.partial(
    pl.kernel,
    out_type=jax.ShapeDtypeStruct((NC, 2 * NP), jnp.float32),
    mesh=_MESH,
    scratch_types=[
        pltpu.VMEM((NWIN_D, WIN), jnp.int32),
        pltpu.VMEM((NWIN_D, WIN), jnp.int32),
        pltpu.VMEM((WIN,), jnp.float32),
        pltpu.VMEM_SHARED((NP,), jnp.float32),
        pltpu.VMEM_SHARED((NP,), jnp.float32),
        pltpu.SemaphoreType.DMA,
    ],
)
def _deg_kernel(src_hbm, dst_hbm, ones_hbm, zdeg_hbm, degp_hbm,
                srci_v, dsti_v, ones_v, dego_sh, degi_sh, dsem):
    cid = lax.axis_index("c")
    sid = lax.axis_index("s")
    tid = cid * NS + sid
    pltpu.sync_copy(zdeg_hbm, dego_sh.at[pl.ds(sid * (NP // NS), NP // NS)])
    pltpu.sync_copy(zdeg_hbm, degi_sh.at[pl.ds(sid * (NP // NS), NP // NS)])
    pltpu.sync_copy(src_hbm.at[tid], srci_v)
    pltpu.sync_copy(dst_hbm.at[tid], dsti_v)
    pltpu.sync_copy(ones_hbm, ones_v)
    plsc.subcore_barrier()

    def batch(bi, carry):
        for q in range(5):
            j = bi * 5 + q
            pltpu.async_copy(ones_v, dego_sh.at[srci_v.at[j]], dsem, add=True)
            pltpu.async_copy(ones_v, degi_sh.at[dsti_v.at[j]], dsem, add=True)
        for q in range(5):
            j = bi * 5 + q
            pltpu.make_async_copy(ones_v, dego_sh.at[srci_v.at[j]],
                                  dsem).wait()
            pltpu.make_async_copy(ones_v, degi_sh.at[dsti_v.at[j]],
                                  dsem).wait()
        return carry

    lax.fori_loop(0, NWIN_D // 5, batch, 0)
    plsc.subcore_barrier()
    pltpu.sync_copy(dego_sh.at[pl.ds(sid * (NP // NS), NP // NS)],
                    degp_hbm.at[cid, pl.ds(sid * (NP // NS), NP // NS)])
    pltpu.sync_copy(degi_sh.at[pl.ds(sid * (NP // NS), NP // NS)],
                    degp_hbm.at[cid, pl.ds(NP + sid * (NP // NS), NP // NS)])


@functools.partial(
    pl.kernel,
    out_type=jax.ShapeDtypeStruct((NC, NP, DH), jnp.float32),
    mesh=_MESH,
    scratch_types=[
        pltpu.VMEM((NWIN_A, WIN), jnp.int32),
        pltpu.VMEM((NWIN_A, WIN), jnp.int32),
        pltpu.VMEM((NWIN_A, WIN), jnp.float32),
        [pltpu.VMEM((WIN, DH), jnp.float32) for _ in range(5)],
        pltpu.VMEM_SHARED((NP, DH), jnp.float32),
        [pltpu.SemaphoreType.DMA for _ in range(5)],
        [pltpu.SemaphoreType.DMA for _ in range(5)],
    ],
    compiler_params=pltpu.CompilerParams(use_tc_tiling_on_sc=False),
)
def _agg_kernel(feat_hbm, src_hbm, dst_hbm, w_hbm, zrows_hbm, parts_hbm,
                src_v, dst_v, w_v, bufs, agg_sh, gsems, ssems):
    cid = lax.axis_index("c")
    sid = lax.axis_index("s")
    pltpu.sync_copy(zrows_hbm, agg_sh.at[pl.ds(sid * ROWS_PT, ROWS_PT)])
    pltpu.sync_copy(src_hbm.at[sid], src_v)
    pltpu.sync_copy(dst_hbm.at[sid], dst_v)
    pltpu.sync_copy(w_hbm.at[sid], w_v)

    coff = jnp.full((16,), cid * NP, dtype=jnp.int32)

    def fix(i, carry):
        j = i // (WIN // 16)
        g = i - j * (WIN // 16)
        sl = pl.ds(g * 16, 16)
        src_v[j, sl] = src_v[j, sl] + coff
        return carry

    def fix3(i, carry):
        return fix(i, carry)

    lax.fori_loop(0, 4 * (WIN // 16), fix3, 0)
    for q in range(4):
        pltpu.async_copy(feat_hbm.at[src_v.at[q]], bufs[q], gsems[q])
    lax.fori_loop(4 * (WIN // 16), NWIN_A * (WIN // 16), fix, 0)
    plsc.subcore_barrier()

    def scale(buf, j):
        for g in range(WIN // 16):
            w16 = w_v[j, pl.ds(g * 16, 16)]
            for l in range(16):
                w = w16[l]
                for k in range(DH // 16):
                    sl = pl.ds(k * 16, 16)
                    buf[g * 16 + l, sl] = buf[g * 16 + l, sl] * w

    def slot(q, base):
        j = base + q
        pltpu.make_async_copy(feat_hbm.at[src_v.at[j]], bufs[q],
                              gsems[q]).wait()
        scale(bufs[q], j)
        pltpu.async_copy(bufs[q], agg_sh.at[dst_v.at[j]], ssems[q], add=True)
        qn = (q + 4) % 5

        @pl.when(j >= 1)
        def _():
            pltpu.make_async_copy(bufs[qn], agg_sh.at[dst_v.at[j - 1]],
                                  ssems[qn]).wait()

        @pl.when(j + 4 < NWIN_A)
        def _():
            pltpu.async_copy(feat_hbm.at[src_v.at[j + 4]], bufs[qn],
                             gsems[qn])

    def block(i, carry):
        base = i * 5
        for q in range(5):
            slot(q, base)
        return carry

    lax.fori_loop(0, NWIN_A // 5, block, 0)
    pltpu.make_async_copy(bufs[4], agg_sh.at[dst_v.at[NWIN_A - 1]],
                          ssems[4]).wait()
    plsc.subcore_barrier()
    pltpu.sync_copy(agg_sh.at[pl.ds(sid * ROWS_PT, ROWS_PT)],
                    parts_hbm.at[cid, pl.ds(sid * ROWS_PT, ROWS_PT)])


BLK = 1280


def _feat_body(x_ref, degp_ref, feat_ref):
    d = degp_ref[...]
    norm = lax.rsqrt(jnp.maximum(d[0] + d[1], 1.0))
    xb = x_ref[...] * norm[:, None]
    feat_ref[0] = xb[:, :DH]
    feat_ref[1] = xb[:, DH:]


_feat_call = pl.pallas_call(
    _feat_body,
    grid=(NP // BLK,),
    in_specs=[
        pl.BlockSpec((BLK, D), lambda i: (i, 0)),
        pl.BlockSpec((2, BLK), lambda i: (0, i)),
    ],
    out_specs=pl.BlockSpec((NC, BLK, DH), lambda i: (0, i, 0)),
    out_shape=jax.ShapeDtypeStruct((NC, NP, DH), jnp.float32),
)


def _out_body(p_ref, w_ref, degp_ref, b_ref, o_ref):
    p = p_ref[...]
    a = jnp.concatenate([p[0], p[1]], axis=1)
    r = jnp.dot(a, w_ref[...], preferred_element_type=jnp.float32)
    d = degp_ref[...]
    norm = lax.rsqrt(jnp.maximum(d[0] + d[1], 1.0))
    o_ref[...] = r * norm[:, None] + b_ref[...]


_out_call = pl.pallas_call(
    _out_body,
    grid=(NP // BLK,),
    in_specs=[
        pl.BlockSpec((NC, BLK, DH), lambda i: (0, i, 0)),
        pl.BlockSpec((D, D), lambda i: (0, 0)),
        pl.BlockSpec((2, BLK), lambda i: (0, NP // BLK + i)),
        pl.BlockSpec((1, D), lambda i: (0, 0)),
    ],
    out_specs=pl.BlockSpec((BLK, D), lambda i: (i, 0)),
    out_shape=jax.ShapeDtypeStruct((NP, D), jnp.float32),
)


def kernel(node_embedding, edge_embedding, edge_index, W, b):
    ei = edge_index.astype(jnp.int32)
    src_d = ei[0].reshape(NW, NWIN_D, WIN)
    dst_d = ei[1].reshape(NW, NWIN_D, WIN)
    src_a = ei[0].reshape(NS, NWIN_A, WIN)
    dst_a = ei[1].reshape(NS, NWIN_A, WIN)
    wts = edge_embedding.astype(jnp.float32).reshape(NS, NWIN_A, WIN)
    ones = jnp.ones((WIN,), jnp.float32)
    zdeg = jnp.zeros((NP // NS,), jnp.float32)
    zrows = jnp.zeros((ROWS_PT, DH), jnp.float32)
    x_pad = jnp.zeros((NP, D), jnp.float32).at[:N].set(node_embedding)

    degp = _deg_kernel(src_d, dst_d, ones, zdeg)
    feat = _feat_call(x_pad, degp).reshape(2 * NP, DH)
    parts = _agg_kernel(feat, src_a, dst_a, wts, zrows)
    return _out_call(parts, W, degp, b.reshape(1, D))[:N]


# speedup vs baseline: 1.3118x; 1.0008x over previous
import functools

import jax
import jax.numpy as jnp
from jax import lax
from jax.experimental import pallas as pl
from jax.experimental.pallas import tpu as pltpu
from jax.experimental.pallas import tpu_sc as plsc

N = 10000
NP = 10240
E = 320000
D = 128
DH = D // 2
NC = 2
NS = 16
NW = NC * NS
WIN = 80
EPW = E // NW
NWIN_D = EPW // WIN
EPS = E // NS
NWIN_A = EPS // WIN
ROWS_PT = NP // NS
DEGW = (2 * NP) // NS

_MESH = plsc.VectorSubcoreMesh(core_axis_name="c", subcore_axis_name="s")


@functools.partial(
    pl.kernel,
    out_type=jax.ShapeDtypeStruct((NC, 2 * NP), jnp.float32),
    mesh=_MESH,
    scratch_types=[
        pltpu.VMEM((NWIN_D, WIN), jnp.int32),
        pltpu.VMEM((NWIN_D, WIN), jnp.int32),
        pltpu.VMEM((WIN,), jnp.float32),
        pltpu.VMEM_SHARED((NP,), jnp.float32),
        pltpu.VMEM_SHARED((NP,), jnp.float32),
        pltpu.SemaphoreType.DMA,
    ],
)
def _deg_kernel(src_hbm, dst_hbm, ones_hbm, zdeg_hbm, degp_hbm,
                srci_v, dsti_v, ones_v, dego_sh, degi_sh, dsem):
    cid = lax.axis_index("c")
    sid = lax.axis_index("s")
    tid = cid * NS + sid
    pltpu.sync_copy(zdeg_hbm, dego_sh.at[pl.ds(sid * (NP // NS), NP // NS)])
    pltpu.sync_copy(zdeg_hbm, degi_sh.at[pl.ds(sid * (NP // NS), NP // NS)])
    pltpu.sync_copy(src_hbm.at[tid], srci_v)
    pltpu.sync_copy(dst_hbm.at[tid], dsti_v)
    pltpu.sync_copy(ones_hbm, ones_v)
    plsc.subcore_barrier()

    def batch(bi, carry):
        for q in range(5):
            j = bi * 5 + q
            pltpu.async_copy(ones_v, dego_sh.at[srci_v.at[j]], dsem, add=True)
            pltpu.async_copy(ones_v, degi_sh.at[dsti_v.at[j]], dsem, add=True)
        for q in range(5):
            j = bi * 5 + q
            pltpu.make_async_copy(ones_v, dego_sh.at[srci_v.at[j]],
                                  dsem).wait()
            pltpu.make_async_copy(ones_v, degi_sh.at[dsti_v.at[j]],
                                  dsem).wait()
        return carry

    lax.fori_loop(0, NWIN_D // 5, batch, 0)
    plsc.subcore_barrier()
    pltpu.sync_copy(dego_sh.at[pl.ds(sid * (NP // NS), NP // NS)],
                    degp_hbm.at[cid, pl.ds(sid * (NP // NS), NP // NS)])
    pltpu.sync_copy(degi_sh.at[pl.ds(sid * (NP // NS), NP // NS)],
                    degp_hbm.at[cid, pl.ds(NP + sid * (NP // NS), NP // NS)])


@functools.partial(
    pl.kernel,
    out_type=jax.ShapeDtypeStruct((NC, NP, DH), jnp.float32),
    mesh=_MESH,
    scratch_types=[
        pltpu.VMEM((NWIN_A, WIN), jnp.int32),
        pltpu.VMEM((NWIN_A, WIN), jnp.int32),
        pltpu.VMEM((NWIN_A, WIN), jnp.float32),
        [pltpu.VMEM((WIN, DH), jnp.float32) for _ in range(5)],
        pltpu.VMEM_SHARED((NP, DH), jnp.float32),
        [pltpu.SemaphoreType.DMA for _ in range(5)],
        [pltpu.SemaphoreType.DMA for _ in range(5)],
    ],
    compiler_params=pltpu.CompilerParams(use_tc_tiling_on_sc=False),
)
def _agg_kernel(feat_hbm, src_hbm, dst_hbm, w_hbm, zrows_hbm, parts_hbm,
                src_v, dst_v, w_v, bufs, agg_sh, gsems, ssems):
    cid = lax.axis_index("c")
    sid = lax.axis_index("s")
    pltpu.sync_copy(zrows_hbm, agg_sh.at[pl.ds(sid * ROWS_PT, ROWS_PT)])
    pltpu.sync_copy(src_hbm.at[sid], src_v)
    pltpu.sync_copy(dst_hbm.at[sid], dst_v)
    pltpu.sync_copy(w_hbm.at[sid], w_v)

    coff = jnp.full((16,), cid * NP, dtype=jnp.int32)

    def fix(i, carry):
        j = i // (WIN // 16)
        g = i - j * (WIN // 16)
        sl = pl.ds(g * 16, 16)
        src_v[j, sl] = src_v[j, sl] + coff
        return carry

    lax.fori_loop(0, 4 * (WIN // 16), fix, 0)
    for q in range(4):
        pltpu.async_copy(feat_hbm.at[src_v.at[q]], bufs[q], gsems[q])
    lax.fori_loop(4 * (WIN // 16), NWIN_A * (WIN // 16), fix, 0)
    plsc.subcore_barrier()

    def scale(buf, j):
        for g in range(WIN // 16):
            w16 = w_v[j, pl.ds(g * 16, 16)]
            for l in range(16):
                w = w16[l]
                for k in range(DH // 16):
                    sl = pl.ds(k * 16, 16)
                    buf[g * 16 + l, sl] = buf[g * 16 + l, sl] * w

    def slot(q, base):
        j = base + q
        pltpu.make_async_copy(feat_hbm.at[src_v.at[j]], bufs[q],
                              gsems[q]).wait()
        scale(bufs[q], j)
        pltpu.async_copy(bufs[q], agg_sh.at[dst_v.at[j]], ssems[q], add=True)
        qn = (q + 4) % 5

        @pl.when(j >= 1)
        def _():
            pltpu.make_async_copy(bufs[qn], agg_sh.at[dst_v.at[j - 1]],
                                  ssems[qn]).wait()

        @pl.when(j + 4 < NWIN_A)
        def _():
            pltpu.async_copy(feat_hbm.at[src_v.at[j + 4]], bufs[qn],
                             gsems[qn])

    def block(i, carry):
        base = i * 5
        for q in range(5):
            slot(q, base)
        return carry

    lax.fori_loop(0, NWIN_A // 5, block, 0)
    pltpu.make_async_copy(bufs[4], agg_sh.at[dst_v.at[NWIN_A - 1]],
                          ssems[4]).wait()
    plsc.subcore_barrier()
    pltpu.sync_copy(agg_sh.at[pl.ds(sid * ROWS_PT, ROWS_PT)],
                    parts_hbm.at[cid, pl.ds(sid * ROWS_PT, ROWS_PT)])


BLK = 1280


def _feat_body(x_ref, degp_ref, feat_ref):
    d = degp_ref[...]
    norm = lax.rsqrt(jnp.maximum(d[0] + d[1], 1.0))
    xb = x_ref[...] * norm[:, None]
    feat_ref[0] = xb[:, :DH]
    feat_ref[1] = xb[:, DH:]


_feat_call = pl.pallas_call(
    _feat_body,
    grid=(NP // BLK,),
    in_specs=[
        pl.BlockSpec((BLK, D), lambda i: (i, 0)),
        pl.BlockSpec((2, BLK), lambda i: (0, i)),
    ],
    out_specs=pl.BlockSpec((NC, BLK, DH), lambda i: (0, i, 0)),
    out_shape=jax.ShapeDtypeStruct((NC, NP, DH), jnp.float32),
)


def _out_body(p_ref, w_ref, degp_ref, b_ref, o_ref):
    p = p_ref[...]
    a = jnp.concatenate([p[0], p[1]], axis=1)
    r = jnp.dot(a, w_ref[...], preferred_element_type=jnp.float32)
    d = degp_ref[...]
    norm = lax.rsqrt(jnp.maximum(d[0] + d[1], 1.0))
    o_ref[...] = r * norm[:, None] + b_ref[...]


_out_call = pl.pallas_call(
    _out_body,
    grid=(NP // BLK,),
    in_specs=[
        pl.BlockSpec((NC, BLK, DH), lambda i: (0, i, 0)),
        pl.BlockSpec((D, D), lambda i: (0, 0)),
        pl.BlockSpec((2, BLK), lambda i: (0, NP // BLK + i)),
        pl.BlockSpec((1, D), lambda i: (0, 0)),
    ],
    out_specs=pl.BlockSpec((BLK, D), lambda i: (i, 0)),
    out_shape=jax.ShapeDtypeStruct((NP, D), jnp.float32),
)


def kernel(node_embedding, edge_embedding, edge_index, W, b):
    ei = edge_index.astype(jnp.int32)
    src_d = ei[0].reshape(NW, NWIN_D, WIN)
    dst_d = ei[1].reshape(NW, NWIN_D, WIN)
    src_a = ei[0].reshape(NS, NWIN_A, WIN)
    dst_a = ei[1].reshape(NS, NWIN_A, WIN)
    wts = edge_embedding.astype(jnp.float32).reshape(NS, NWIN_A, WIN)
    ones = jnp.ones((WIN,), jnp.float32)
    zdeg = jnp.zeros((NP // NS,), jnp.float32)
    zrows = jnp.zeros((ROWS_PT, DH), jnp.float32)
    x_pad = jnp.zeros((NP, D), jnp.float32).at[:N].set(node_embedding)

    degp = _deg_kernel(src_d, dst_d, ones, zdeg)
    feat = _feat_call(x_pad, degp).reshape(2 * NP, DH)
    parts = _agg_kernel(feat, src_a, dst_a, wts, zrows)
    return _out_call(parts, W, degp, b.reshape(1, D))[:N]
